# baseline (speedup 1.0000x reference)
import functools

import jax
import jax.numpy as jnp
import numpy as np
from jax import lax
from jax.experimental import pallas as pl
from jax.experimental.pallas import tpu as pltpu
from jax.experimental.pallas import tpu_sc as plsc

N = 10000
E = 320000
D = 128
DE = 16
H = 64
C = 40
G = 64
NEG_SLOPE = 0.2

NC = 2
NS = 16
NW = NC * NS
EW = E // NW
EB = 40
NCH = EW // EB
NP = 10240
RPT = NP // NS


def _perm(width):
    p = np.zeros(width, np.int64)
    for j in range(width // 32):
        for i in range(16):
            for k in range(2):
                p[32 * j + 2 * i + k] = 32 * j + 16 * k + i
    return p


P128 = _perm(D)
P64 = _perm(H)


def _edge_stats_body(ea_ref, out_ref):
    @pl.when(pl.program_id(0) == 0)
    def _():
        out_ref[...] = jnp.zeros_like(out_ref)
    blk = ea_ref[...]
    s0 = jnp.sum(blk, axis=0)
    s1 = jnp.sum(blk * blk, axis=0)
    out_ref[...] += jnp.stack([s0, s1])


def _edge_stats(edge_attr):
    BE = 8000
    return pl.pallas_call(
        _edge_stats_body,
        grid=(E // BE,),
        in_specs=[pl.BlockSpec((BE, DE), lambda i: (i, 0))],
        out_specs=pl.BlockSpec((2, DE), lambda i: (0, 0)),
        out_shape=jax.ShapeDtypeStruct((2, DE), jnp.float32),
    )(edge_attr)


def _node_bn_body(x_ref, g_ref, b_ref, out_ref):
    x = x_ref[...]
    mean = jnp.mean(x, axis=0, keepdims=True)
    var = jnp.var(x, axis=0, keepdims=True)
    out_ref[...] = (x - mean) / jnp.sqrt(var + 1e-5) * g_ref[...] + b_ref[...]


def _node_bn(x, node_g, node_b):
    return pl.pallas_call(
        _node_bn_body,
        out_shape=jax.ShapeDtypeStruct((N, D), jnp.float32),
    )(x, node_g.reshape(1, D), node_b.reshape(1, D))


def _pack_bf16(y):
    r, f = y.shape
    yb = y.astype(jnp.bfloat16).astype(jnp.float32)
    bits = lax.bitcast_convert_type(yb, jnp.int32).reshape(r, f // 32, 32)
    lo = lax.shift_right_logical(bits[:, :, :16], 16)
    hi = lax.bitwise_and(bits[:, :, 16:], jnp.int32(-65536))
    return lax.bitwise_or(lo, hi).reshape(r, f // 2)


def _edge_mm_body(ea_ref, w0_ref, b0_ref, w1_ref, b1_ref, w2_ref, b2_ref,
                  e0_ref, e1_ref, e2_ref):
    ea = ea_ref[...]
    e0_ref[...] = _pack_bf16(
        jnp.dot(ea, w0_ref[...], preferred_element_type=jnp.float32) + b0_ref[...])
    e1_ref[...] = _pack_bf16(
        jnp.dot(ea, w1_ref[...], preferred_element_type=jnp.float32) + b1_ref[...])
    e2_ref[...] = _pack_bf16(
        jnp.dot(ea, w2_ref[...], preferred_element_type=jnp.float32) + b2_ref[...])


def _edge_mm(edge_attr, w0, b0, w1, b1, w2, b2):
    BE = 8000
    full = lambda r, c: pl.BlockSpec((r, c), lambda i: (0, 0))
    return pl.pallas_call(
        _edge_mm_body,
        grid=(E // BE,),
        in_specs=[pl.BlockSpec((BE, DE), lambda i: (i, 0)),
                  full(DE, D), full(1, D), full(DE, H), full(1, H),
                  full(DE, H), full(1, H)],
        out_specs=[pl.BlockSpec((BE, D // 2), lambda i: (i, 0)),
                   pl.BlockSpec((BE, H // 2), lambda i: (i, 0)),
                   pl.BlockSpec((BE, H // 2), lambda i: (i, 0))],
        out_shape=[jax.ShapeDtypeStruct((E, D // 2), jnp.int32),
                   jax.ShapeDtypeStruct((E, H // 2), jnp.int32),
                   jax.ShapeDtypeStruct((E, H // 2), jnp.int32)],
    )(edge_attr, w0, b0.reshape(1, D), w1, b1.reshape(1, H),
      w2, b2.reshape(1, H))


def _mlp_body(din, fa, pad_out, x_ref, a0_ref, a1_ref, eps_ref, w1_ref,
              b1_ref, w2_ref, b2_ref, out_ref):
    h = ((1.0 + eps_ref[0, 0]) * x_ref[...][:, :din]
         + a0_ref[0][:, :din] + a1_ref[0][:, :din])
    h = jnp.maximum(jnp.dot(h, w1_ref[...], preferred_element_type=jnp.float32)
                    + b1_ref[...], 0.0)
    y = jnp.dot(h, w2_ref[...], preferred_element_type=jnp.float32) + b2_ref[...]
    y = jnp.where(y >= 0.0, y, NEG_SLOPE * y)
    if pad_out:
        y = jnp.concatenate([y, jnp.zeros_like(y)], axis=1)
    out_ref[...] = y


def _mlp(x, aggr, eps, w1, b1, w2, b2, din, fa, pad_out):
    BN = 2000
    dout = 2 * H if pad_out else H
    full = lambda r, c: pl.BlockSpec((r, c), lambda i: (0, 0))
    return pl.pallas_call(
        functools.partial(_mlp_body, din, fa, pad_out),
        grid=(N // BN,),
        in_specs=[pl.BlockSpec((BN, D), lambda i: (i, 0)),
                  pl.BlockSpec((1, BN, fa), lambda i: (0, i, 0)),
                  pl.BlockSpec((1, BN, fa), lambda i: (1, i, 0)),
                  full(1, 1), full(din, H), full(1, H),
                  full(H, H), full(1, H)],
        out_specs=pl.BlockSpec((BN, dout), lambda i: (i, 0)),
        out_shape=jax.ShapeDtypeStruct((N, dout), jnp.float32),
    )(x, aggr, aggr, eps.reshape(1, 1), w1, b1.reshape(1, H),
      w2, b2.reshape(1, H))


def _seg_max_body(x_ref, b_ref, t_ref, out_ref):
    @pl.when(pl.program_id(0) == 0)
    def _():
        out_ref[...] = jnp.full_like(out_ref, -3e38)
    rows = x_ref[...] * t_ref[0, 0]
    bc = b_ref[...]
    m3 = bc[:, :, None] == lax.broadcasted_iota(jnp.int32, (1, G, 1), 1)
    vals = jnp.where(m3, rows[:, None, :], -3e38)
    out_ref[...] = jnp.maximum(out_ref[...], jnp.max(vals, axis=0))


def _seg_max(x, batch2, t):
    CH = 400
    return pl.pallas_call(
        _seg_max_body,
        grid=(N // CH,),
        in_specs=[pl.BlockSpec((CH, H), lambda i: (i, 0)),
                  pl.BlockSpec((CH, 1), lambda i: (i, 0)),
                  pl.BlockSpec((1, 1), lambda i: (0, 0))],
        out_specs=pl.BlockSpec((G, H), lambda i: (0, 0)),
        out_shape=jax.ShapeDtypeStruct((G, H), jnp.float32),
    )(x, batch2, t)


def _readout_body(x_ref, b_ref, t_ref, wl_ref, bl_ref, smax_ref, out_ref):
    xv = x_ref[...]
    s = xv * t_ref[0, 0]
    b = b_ref[...]
    onehot = (b == lax.broadcasted_iota(jnp.int32, (1, G), 1)).astype(jnp.float32)
    smax = smax_ref[...]
    sm_n = jnp.dot(onehot, smax, preferred_element_type=jnp.float32)
    es = jnp.exp(s - sm_n)
    denom = lax.dot_general(onehot, es, (((0,), (0,)), ((), ())),
                            preferred_element_type=jnp.float32)
    den_n = jnp.dot(onehot, denom, preferred_element_type=jnp.float32)
    alpha = es / (den_n + 1e-16)
    seg = lax.dot_general(onehot, alpha * xv, (((0,), (0,)), ((), ())),
                          preferred_element_type=jnp.float32)
    out_ref[...] = jnp.dot(seg, wl_ref[...], preferred_element_type=jnp.float32) + bl_ref[...]


def _readout(x, batch, t, wl, bl):
    batch2 = batch.reshape(N, 1)
    t2 = t.reshape(1, 1)
    smax = _seg_max(x, batch2, t2)
    return pl.pallas_call(
        _readout_body,
        out_shape=jax.ShapeDtypeStruct((G, C), jnp.float32),
    )(x, batch2, t2, wl, bl.reshape(1, C), smax)


@functools.partial(jax.jit, static_argnames=("FE",))
def _sc_edge(x, src3, dst3, e, zrows, FE):
    mesh = plsc.VectorSubcoreMesh(core_axis_name="c", subcore_axis_name="s")
    inplace = True

    @functools.partial(
        pl.kernel,
        out_type=jax.ShapeDtypeStruct((NC, NP, D), jnp.float32),
        mesh=mesh,
        compiler_params=pltpu.CompilerParams(needs_layout_passes=False),
        scratch_types=[
            pltpu.VMEM((2, EB), jnp.int32),
            pltpu.VMEM((2, EB), jnp.int32),
            pltpu.VMEM((2, EB, D), jnp.float32),
            pltpu.VMEM((2, EB, FE // 2), jnp.int32),
        ] + ([] if inplace else [pltpu.VMEM((EB, FE), jnp.float32)]) + [
            pltpu.VMEM_SHARED((NP, D), jnp.float32),
            pltpu.SemaphoreType.DMA,
            pltpu.SemaphoreType.DMA,
            pltpu.SemaphoreType.DMA,
            pltpu.SemaphoreType.DMA,
        ],
    )
    def k(x_hbm, src_hbm, dst_hbm, e_hbm, z_hbm, out_hbm,
          src_v, dst_v, xg_v, e_v, *rest):
        if inplace:
            m_v = None
            aggr_sh, sg0, sg1, se0, se1 = rest
        else:
            m_v, aggr_sh, sg0, sg1, se0, se1 = rest
        c = lax.axis_index("c")
        s = lax.axis_index("s")
        wid = c * NS + s
        pltpu.sync_copy(z_hbm, aggr_sh.at[pl.ds(s * RPT, RPT)])
        plsc.subcore_barrier()

        ebase = wid * EW
        sems = ((sg0, se0), (sg1, se1))

        def issue(ch, b):
            pltpu.sync_copy(src_hbm.at[wid, ch], src_v.at[b])
            pltpu.sync_copy(dst_hbm.at[wid, ch], dst_v.at[b])
            pltpu.async_copy(e_hbm.at[pl.ds(ebase + ch * EB, EB)], e_v.at[b],
                             sems[b][1])
            pltpu.async_copy(x_hbm.at[src_v.at[b]], xg_v.at[b], sems[b][0])

        issue(0, 0)
        issue(1, 1)

        def halfstep(ch, b):
            pltpu.make_async_copy(e_hbm.at[pl.ds(0, EB)], e_v.at[b],
                                  sems[b][1]).wait()
            pltpu.make_async_copy(x_hbm.at[src_v.at[b]], xg_v.at[b],
                                  sems[b][0]).wait()

            def row(r, carry2):
                for j in range(FE // 32):
                    ew = plsc.bitcast(e_v[b, r, pl.ds(j * 16, 16)], jnp.bfloat16)
                    lo, hi = plsc.unpack(ew, format=plsc.PackFormat.INTERLEAVED,
                                         preferred_element_type=jnp.float32)
                    s0 = pl.ds(j * 32, 16)
                    s1 = pl.ds(j * 32 + 16, 16)
                    xg_v[b, r, s0] = jnp.maximum(xg_v[b, r, s0] + lo, 0.0)
                    xg_v[b, r, s1] = jnp.maximum(xg_v[b, r, s1] + hi, 0.0)
                return carry2

            lax.fori_loop(0, EB, row, 0)
            if inplace:
                pltpu.sync_copy(xg_v.at[b], aggr_sh.at[dst_v.at[b]], add=True)
            else:
                pltpu.sync_copy(m_v, aggr_sh.at[dst_v.at[b]], add=True)

            @pl.when(ch + 2 < NCH)
            def _():
                issue(ch + 2, b)

        def body(j, carry):
            halfstep(2 * j, 0)
            halfstep(2 * j + 1, 1)
            return carry

        lax.fori_loop(0, NCH // 2, body, 0)
        plsc.subcore_barrier()
        pltpu.sync_copy(aggr_sh.at[pl.ds(s * RPT, RPT)],
                        out_hbm.at[c, pl.ds(s * RPT, RPT)])

    return k(x, src3, dst3, e, zrows)


def kernel(x, edge_index, edge_attr, batch, node_g, node_b, edge_g, edge_b,
           eps0, We0, be0, W10, b10, W20, b20,
           eps1, We1, be1, W11, b11, W21, b21,
           eps2, We2, be2, W12, b12, W22, b22,
           t, Wl, bl):
    src3 = edge_index[0].reshape(NW, NCH, EB)
    dst3 = edge_index[1].reshape(NW, NCH, EB)

    stats = _edge_stats(edge_attr)
    mean_e = stats[0] / E
    var_e = stats[1] / E - mean_e * mean_e
    scale = edge_g / jnp.sqrt(var_e + 1e-5)
    shift = edge_b - mean_e * scale
    Wp = [We * scale[:, None] for We in (We0, We1, We2)]
    bp = [shift @ We + be for We, be in ((We0, be0), (We1, be1), (We2, be2))]
    e0, e1, e2 = _edge_mm(edge_attr, Wp[0], bp[0], Wp[1], bp[1], Wp[2], bp[2])

    xc = _node_bn(x, node_g, node_b)

    z128 = jnp.zeros((RPT, D), jnp.float32)
    z64 = jnp.zeros((RPT, H), jnp.float32)

    aggr = _sc_edge(xc, src3, dst3, e0, z128, FE=D)
    xc = _mlp(xc, aggr, eps0, W10, b10, W20, b20, din=D, fa=D, pad_out=True)

    aggr = _sc_edge(xc, src3, dst3, e1, z128, FE=H)
    xc = _mlp(xc, aggr, eps1, W11, b11, W21, b21, din=H, fa=D, pad_out=True)

    aggr = _sc_edge(xc, src3, dst3, e2, z128, FE=H)
    xc = _mlp(xc, aggr, eps2, W12, b12, W22, b22, din=H, fa=D, pad_out=False)

    return _readout(xc, batch, t, Wl, bl)


# speedup vs baseline: 1.6796x; 1.6796x over previous
import functools

import jax
import jax.numpy as jnp
import numpy as np
from jax import lax
from jax.experimental import pallas as pl
from jax.experimental.pallas import tpu as pltpu
from jax.experimental.pallas import tpu_sc as plsc

N = 10000
E = 320000
D = 128
DE = 16
H = 64
C = 40
G = 64
NEG_SLOPE = 0.2

NC = 2
NS = 16
NW = NC * NS
EW = E // NW
EB = 40
NCH = EW // EB
NP = 10240
RPT = NP // NS


def _perm(width):
    p = np.zeros(width, np.int64)
    for j in range(width // 32):
        for i in range(16):
            for k in range(2):
                p[32 * j + 2 * i + k] = 32 * j + 16 * k + i
    return p


P128 = _perm(D)
P64 = _perm(H)


def _edge_stats_body(ea_ref, out_ref):
    @pl.when(pl.program_id(0) == 0)
    def _():
        out_ref[...] = jnp.zeros_like(out_ref)
    blk = ea_ref[...]
    s0 = jnp.sum(blk, axis=0)
    s1 = jnp.sum(blk * blk, axis=0)
    out_ref[...] += jnp.stack([s0, s1])


def _edge_stats(edge_attr):
    BE = 8000
    return pl.pallas_call(
        _edge_stats_body,
        grid=(E // BE,),
        in_specs=[pl.BlockSpec((BE, DE), lambda i: (i, 0))],
        out_specs=pl.BlockSpec((2, DE), lambda i: (0, 0)),
        out_shape=jax.ShapeDtypeStruct((2, DE), jnp.float32),
    )(edge_attr)


def _node_bn_body(x_ref, g_ref, b_ref, out_ref):
    x = x_ref[...]
    mean = jnp.mean(x, axis=0, keepdims=True)
    var = jnp.var(x, axis=0, keepdims=True)
    out_ref[...] = (x - mean) / jnp.sqrt(var + 1e-5) * g_ref[...] + b_ref[...]


def _node_bn(x, node_g, node_b):
    return pl.pallas_call(
        _node_bn_body,
        out_shape=jax.ShapeDtypeStruct((N, D), jnp.float32),
    )(x, node_g.reshape(1, D), node_b.reshape(1, D))


def _pack_bf16(y):
    r, f = y.shape
    yb = y.astype(jnp.bfloat16).astype(jnp.float32)
    bits = lax.bitcast_convert_type(yb, jnp.int32).reshape(r, f // 32, 32)
    lo = lax.shift_right_logical(bits[:, :, :16], 16)
    hi = lax.bitwise_and(bits[:, :, 16:], jnp.int32(-65536))
    return lax.bitwise_or(lo, hi).reshape(r, f // 2)


def _edge_mm_body(ea_ref, w0_ref, b0_ref, w1_ref, b1_ref, w2_ref, b2_ref,
                  e0_ref, e1_ref, e2_ref):
    ea = ea_ref[...]
    e0_ref[...] = jnp.dot(ea, w0_ref[...], preferred_element_type=jnp.float32) + b0_ref[...]
    e1_ref[...] = jnp.dot(ea, w1_ref[...], preferred_element_type=jnp.float32) + b1_ref[...]
    e2_ref[...] = jnp.dot(ea, w2_ref[...], preferred_element_type=jnp.float32) + b2_ref[...]


def _edge_mm(edge_attr, w0, b0, w1, b1, w2, b2):
    BE = 8000
    full = lambda r, c: pl.BlockSpec((r, c), lambda i: (0, 0))
    return pl.pallas_call(
        _edge_mm_body,
        grid=(E // BE,),
        in_specs=[pl.BlockSpec((BE, DE), lambda i: (i, 0)),
                  full(DE, D), full(1, D), full(DE, H), full(1, H),
                  full(DE, H), full(1, H)],
        out_specs=[pl.BlockSpec((BE, D), lambda i: (i, 0)),
                   pl.BlockSpec((BE, H), lambda i: (i, 0)),
                   pl.BlockSpec((BE, H), lambda i: (i, 0))],
        out_shape=[jax.ShapeDtypeStruct((E, D), jnp.float32),
                   jax.ShapeDtypeStruct((E, H), jnp.float32),
                   jax.ShapeDtypeStruct((E, H), jnp.float32)],
    )(edge_attr, w0, b0.reshape(1, D), w1, b1.reshape(1, H),
      w2, b2.reshape(1, H))


def _mlp_body(din, fa, pad_out, x_ref, a0_ref, a1_ref, eps_ref, w1_ref,
              b1_ref, w2_ref, b2_ref, out_ref):
    h = ((1.0 + eps_ref[0, 0]) * x_ref[...][:, :din]
         + a0_ref[0][:, :din] + a1_ref[0][:, :din])
    h = jnp.maximum(jnp.dot(h, w1_ref[...], preferred_element_type=jnp.float32)
                    + b1_ref[...], 0.0)
    y = jnp.dot(h, w2_ref[...], preferred_element_type=jnp.float32) + b2_ref[...]
    y = jnp.where(y >= 0.0, y, NEG_SLOPE * y)
    if pad_out:
        y = jnp.concatenate([y, jnp.zeros_like(y)], axis=1)
    out_ref[...] = y


def _mlp(x, aggr, eps, w1, b1, w2, b2, din, fa, pad_out):
    BN = 2000
    dout = 2 * H if pad_out else H
    full = lambda r, c: pl.BlockSpec((r, c), lambda i: (0, 0))
    return pl.pallas_call(
        functools.partial(_mlp_body, din, fa, pad_out),
        grid=(N // BN,),
        in_specs=[pl.BlockSpec((BN, D), lambda i: (i, 0)),
                  pl.BlockSpec((1, BN, fa), lambda i: (0, i, 0)),
                  pl.BlockSpec((1, BN, fa), lambda i: (1, i, 0)),
                  full(1, 1), full(din, H), full(1, H),
                  full(H, H), full(1, H)],
        out_specs=pl.BlockSpec((BN, dout), lambda i: (i, 0)),
        out_shape=jax.ShapeDtypeStruct((N, dout), jnp.float32),
    )(x, aggr, aggr, eps.reshape(1, 1), w1, b1.reshape(1, H),
      w2, b2.reshape(1, H))


def _seg_max_body(x_ref, b_ref, t_ref, out_ref):
    @pl.when(pl.program_id(0) == 0)
    def _():
        out_ref[...] = jnp.full_like(out_ref, -3e38)
    rows = x_ref[...] * t_ref[0, 0]
    bc = b_ref[...]
    m3 = bc[:, :, None] == lax.broadcasted_iota(jnp.int32, (1, G, 1), 1)
    vals = jnp.where(m3, rows[:, None, :], -3e38)
    out_ref[...] = jnp.maximum(out_ref[...], jnp.max(vals, axis=0))


def _seg_max(x, batch2, t):
    CH = 400
    return pl.pallas_call(
        _seg_max_body,
        grid=(N // CH,),
        in_specs=[pl.BlockSpec((CH, H), lambda i: (i, 0)),
                  pl.BlockSpec((CH, 1), lambda i: (i, 0)),
                  pl.BlockSpec((1, 1), lambda i: (0, 0))],
        out_specs=pl.BlockSpec((G, H), lambda i: (0, 0)),
        out_shape=jax.ShapeDtypeStruct((G, H), jnp.float32),
    )(x, batch2, t)


def _readout_body(x_ref, b_ref, t_ref, wl_ref, bl_ref, smax_ref, out_ref):
    xv = x_ref[...]
    s = xv * t_ref[0, 0]
    b = b_ref[...]
    onehot = (b == lax.broadcasted_iota(jnp.int32, (1, G), 1)).astype(jnp.float32)
    smax = smax_ref[...]
    sm_n = jnp.dot(onehot, smax, preferred_element_type=jnp.float32)
    es = jnp.exp(s - sm_n)
    denom = lax.dot_general(onehot, es, (((0,), (0,)), ((), ())),
                            preferred_element_type=jnp.float32)
    den_n = jnp.dot(onehot, denom, preferred_element_type=jnp.float32)
    alpha = es / (den_n + 1e-16)
    seg = lax.dot_general(onehot, alpha * xv, (((0,), (0,)), ((), ())),
                          preferred_element_type=jnp.float32)
    out_ref[...] = jnp.dot(seg, wl_ref[...], preferred_element_type=jnp.float32) + bl_ref[...]


def _readout(x, batch, t, wl, bl):
    batch2 = batch.reshape(N, 1)
    t2 = t.reshape(1, 1)
    smax = _seg_max(x, batch2, t2)
    return pl.pallas_call(
        _readout_body,
        out_shape=jax.ShapeDtypeStruct((G, C), jnp.float32),
    )(x, batch2, t2, wl, bl.reshape(1, C), smax)


@functools.partial(jax.jit, static_argnames=("FE",))
def _sc_edge(x, src3, dst3, e, zrows, FE):
    mesh = plsc.VectorSubcoreMesh(core_axis_name="c", subcore_axis_name="s")
    inplace = True

    @functools.partial(
        pl.kernel,
        out_type=jax.ShapeDtypeStruct((NC, NP, D), jnp.float32),
        mesh=mesh,
        scratch_types=[
            pltpu.VMEM((2, EB), jnp.int32),
            pltpu.VMEM((2, EB), jnp.int32),
            pltpu.VMEM((2, EB, D), jnp.float32),
            pltpu.VMEM((2, EB, FE), jnp.float32),
        ] + ([] if inplace else [pltpu.VMEM((EB, FE), jnp.float32)]) + [
            pltpu.VMEM_SHARED((NP, D), jnp.float32),
            pltpu.SemaphoreType.DMA,
            pltpu.SemaphoreType.DMA,
            pltpu.SemaphoreType.DMA,
            pltpu.SemaphoreType.DMA,
        ],
    )
    def k(x_hbm, src_hbm, dst_hbm, e_hbm, z_hbm, out_hbm,
          src_v, dst_v, xg_v, e_v, *rest):
        if inplace:
            m_v = None
            aggr_sh, sg0, sg1, se0, se1 = rest
        else:
            m_v, aggr_sh, sg0, sg1, se0, se1 = rest
        c = lax.axis_index("c")
        s = lax.axis_index("s")
        wid = c * NS + s
        pltpu.sync_copy(z_hbm, aggr_sh.at[pl.ds(s * RPT, RPT)])
        plsc.subcore_barrier()

        ebase = wid * EW
        sems = ((sg0, se0), (sg1, se1))

        def issue(ch, b):
            pltpu.sync_copy(src_hbm.at[wid, ch], src_v.at[b])
            pltpu.sync_copy(dst_hbm.at[wid, ch], dst_v.at[b])
            pltpu.async_copy(e_hbm.at[pl.ds(ebase + ch * EB, EB)], e_v.at[b],
                             sems[b][1])
            pltpu.async_copy(x_hbm.at[src_v.at[b]], xg_v.at[b], sems[b][0])

        issue(0, 0)
        issue(1, 1)

        def halfstep(ch, b):
            pltpu.make_async_copy(e_hbm.at[pl.ds(0, EB)], e_v.at[b],
                                  sems[b][1]).wait()
            pltpu.make_async_copy(x_hbm.at[src_v.at[b]], xg_v.at[b],
                                  sems[b][0]).wait()

            def row(r, carry2):
                for j in range(FE // 16):
                    sl = pl.ds(j * 16, 16)
                    v = jnp.maximum(xg_v[b, r, sl] + e_v[b, r, sl], 0.0)
                    if inplace:
                        xg_v[b, r, sl] = v
                    else:
                        m_v[r, sl] = v
                return carry2

            lax.fori_loop(0, EB, row, 0)
            if inplace:
                pltpu.sync_copy(xg_v.at[b], aggr_sh.at[dst_v.at[b]], add=True)
            else:
                pltpu.sync_copy(m_v, aggr_sh.at[dst_v.at[b]], add=True)

            @pl.when(ch + 2 < NCH)
            def _():
                issue(ch + 2, b)

        def body(j, carry):
            halfstep(2 * j, 0)
            halfstep(2 * j + 1, 1)
            return carry

        lax.fori_loop(0, NCH // 2, body, 0)
        plsc.subcore_barrier()
        pltpu.sync_copy(aggr_sh.at[pl.ds(s * RPT, RPT)],
                        out_hbm.at[c, pl.ds(s * RPT, RPT)])

    return k(x, src3, dst3, e, zrows)


def kernel(x, edge_index, edge_attr, batch, node_g, node_b, edge_g, edge_b,
           eps0, We0, be0, W10, b10, W20, b20,
           eps1, We1, be1, W11, b11, W21, b21,
           eps2, We2, be2, W12, b12, W22, b22,
           t, Wl, bl):
    src3 = edge_index[0].reshape(NW, NCH, EB)
    dst3 = edge_index[1].reshape(NW, NCH, EB)

    stats = _edge_stats(edge_attr)
    mean_e = stats[0] / E
    var_e = stats[1] / E - mean_e * mean_e
    scale = edge_g / jnp.sqrt(var_e + 1e-5)
    shift = edge_b - mean_e * scale
    Wp = [We * scale[:, None] for We in (We0, We1, We2)]
    bp = [shift @ We + be for We, be in ((We0, be0), (We1, be1), (We2, be2))]
    e0, e1, e2 = _edge_mm(edge_attr, Wp[0], bp[0], Wp[1], bp[1], Wp[2], bp[2])

    xc = _node_bn(x, node_g, node_b)

    z128 = jnp.zeros((RPT, D), jnp.float32)
    z64 = jnp.zeros((RPT, H), jnp.float32)

    aggr = _sc_edge(xc, src3, dst3, e0, z128, FE=D)
    xc = _mlp(xc, aggr, eps0, W10, b10, W20, b20, din=D, fa=D, pad_out=True)

    aggr = _sc_edge(xc, src3, dst3, e1, z128, FE=H)
    xc = _mlp(xc, aggr, eps1, W11, b11, W21, b21, din=H, fa=D, pad_out=True)

    aggr = _sc_edge(xc, src3, dst3, e2, z128, FE=H)
    xc = _mlp(xc, aggr, eps2, W12, b12, W22, b22, din=H, fa=D, pad_out=False)

    return _readout(xc, batch, t, Wl, bl)
